# Initial kernel scaffold; baseline (speedup 1.0000x reference)
import jax
import jax.numpy as jnp
from jax.experimental import pallas as pl


def kernel(x, edge_index, W1, b1, W2, b2, Wf1, bf1, Wf2, bf2):
    raise NotImplementedError("write your pallas kernel here")


# speedup vs baseline: 17.8995x; 17.8995x over previous
import functools

import jax
import jax.numpy as jnp
from jax import lax
from jax.experimental import pallas as pl
from jax.experimental.pallas import tpu as pltpu
from jax.experimental.pallas import tpu_sc as plsc

N = 100000
NC = 2
NS = 16
LANE = 16
NQ = 4
QUARTER = N // NQ
QTROWS = 1568
QPAD = NS * QTROWS
ZROWS = 392
K = 8
CHUNK = K * 128

_MESH = plsc.VectorSubcoreMesh(
    core_axis_name="c", subcore_axis_name="s", num_cores=NC, num_subcores=NS
)
_SC_PARAMS = pltpu.CompilerParams(use_tc_tiling_on_sc=False)


def _localize(sbuf, dbuf, smbuf, dmbuf, base):
  for j in range(K):
    for l in range(128 // LANE):
      sl = pl.ds(l * LANE, LANE)
      dl = dbuf[j, sl] - base
      m = dl.astype(jnp.uint32) < jnp.uint32(QUARTER)
      smbuf[j, sl] = jnp.where(m, sbuf[j, sl], -1)
      dmbuf[j, sl] = jnp.where(m, dl, -1)


def _make_deg_kernel(nrows_total):
  rpt = nrows_total // NS

  def body(dst2, zeros1, deg_out, dbuf, dmbuf, ones, zbuf, acc, gsem):
    c = lax.axis_index("c")
    s = lax.axis_index("s")
    for l in range(128 // LANE):
      ones[pl.ds(l * LANE, LANE)] = jnp.ones((LANE,), jnp.float32)
    pltpu.sync_copy(zeros1, zbuf)
    for p in range(NQ // NC):
      q = p * NC + c
      base = q * QUARTER
      pltpu.sync_copy(zbuf, acc.at[pl.ds(s * QTROWS, QTROWS)])
      plsc.subcore_barrier()

      def chunk(ch, _):
        row0 = s * rpt + ch * K
        pltpu.sync_copy(dst2.at[pl.ds(row0, K)], dbuf)
        for j in range(K):
          for l in range(128 // LANE):
            sl = pl.ds(l * LANE, LANE)
            dl = dbuf[j, sl] - base
            m = dl.astype(jnp.uint32) < jnp.uint32(QUARTER)
            dmbuf[j, sl] = jnp.where(m, dl, -1)
        cps = []
        for j in range(K):
          cps.append(pltpu.async_copy(
              ones, acc.at[plsc.Indices(dmbuf.at[j], ignored_value=-1)],
              gsem, add=True))
        for cp in cps:
          cp.wait()
        return _

      lax.fori_loop(0, rpt // K, chunk, 0)
      plsc.subcore_barrier()
      off = pl.multiple_of(q * QPAD + s * QTROWS, 8)
      pltpu.sync_copy(acc.at[pl.ds(s * QTROWS, QTROWS)], zbuf)
      pltpu.sync_copy(zbuf, deg_out.at[pl.ds(off, QTROWS)])
      if p == 0:
        pltpu.sync_copy(zeros1, zbuf)

  return pl.kernel(
      body,
      out_type=jax.ShapeDtypeStruct((NQ * QPAD,), jnp.float32),
      mesh=_MESH,
      compiler_params=_SC_PARAMS,
      scratch_types=[
          pltpu.VMEM((K, 128), jnp.int32),
          pltpu.VMEM((K, 128), jnp.int32),
          pltpu.VMEM((128,), jnp.float32),
          pltpu.VMEM((QTROWS,), jnp.float32),
          pltpu.VMEM_SHARED((QPAD,), jnp.float32),
          pltpu.SemaphoreType.DMA,
      ],
  )


def _make_scatter_kernel(nrows_total, feat):
  rpt = nrows_total // NS

  def body(src2, dst2, y, zeros2, z_out, sbuf, dbuf, smbuf, dmbuf, zbuf, rows,
           acc, gsem, ssem):
    c = lax.axis_index("c")
    s = lax.axis_index("s")
    for p in range(NQ // NC):
      q = p * NC + c
      base = q * QUARTER
      pltpu.sync_copy(zeros2, zbuf)
      for r in range(QTROWS // ZROWS):
        pltpu.sync_copy(zbuf, acc.at[pl.ds(s * QTROWS + r * ZROWS, ZROWS)])
      plsc.subcore_barrier()

      def chunk(ch, _):
        row0 = s * rpt + ch * K
        pltpu.sync_copy(src2.at[pl.ds(row0, K)], sbuf)
        pltpu.sync_copy(dst2.at[pl.ds(row0, K)], dbuf)
        _localize(sbuf, dbuf, smbuf, dmbuf, base)
        cps = []
        for j in range(K):
          cps.append(pltpu.async_copy(
              y.at[plsc.Indices(smbuf.at[j], ignored_value=-1)],
              rows.at[pl.ds(j * 128, 128)], gsem))
        for cp in cps:
          cp.wait()
        cps = []
        for j in range(K):
          cps.append(pltpu.async_copy(
              rows.at[pl.ds(j * 128, 128)],
              acc.at[plsc.Indices(dmbuf.at[j], ignored_value=-1)],
              ssem, add=True))
        for cp in cps:
          cp.wait()
        return _

      lax.fori_loop(0, rpt // K, chunk, 0)
      plsc.subcore_barrier()
      off = pl.multiple_of(q * QPAD + s * QTROWS, 8)
      for r in range(QTROWS // ZROWS):
        pltpu.sync_copy(acc.at[pl.ds(s * QTROWS + r * ZROWS, ZROWS)], zbuf)
        pltpu.sync_copy(zbuf, z_out.at[pl.ds(off + r * ZROWS, ZROWS)])

  return pl.kernel(
      body,
      out_type=jax.ShapeDtypeStruct((NQ * QPAD, feat), jnp.float32),
      mesh=_MESH,
      compiler_params=_SC_PARAMS,
      scratch_types=[
          pltpu.VMEM((K, 128), jnp.int32),
          pltpu.VMEM((K, 128), jnp.int32),
          pltpu.VMEM((K, 128), jnp.int32),
          pltpu.VMEM((K, 128), jnp.int32),
          pltpu.VMEM((ZROWS, feat), jnp.float32),
          pltpu.VMEM((CHUNK, feat), jnp.float32),
          pltpu.VMEM_SHARED((QPAD, feat), jnp.float32),
          pltpu.SemaphoreType.DMA,
          pltpu.SemaphoreType.DMA,
      ],
  )


_BN = 8192


def _tc1_body(deg_ref, x_ref, d_ref, dx_ref):
  d = lax.rsqrt(deg_ref[...] + 1.0)
  d_ref[...] = d
  dx_ref[...] = x_ref[...] * d


def _tc2_body(z1_ref, x_ref, d_ref, w1_ref, b1_ref, w2_ref, xw2_ref, y2_ref):
  d = d_ref[...]
  u = d * z1_ref[...] + (d * d) * x_ref[...]
  h1 = jnp.maximum(jnp.dot(u, w1_ref[...],
                           preferred_element_type=jnp.float32) + b1_ref[...],
                   0.0)
  xw2 = jnp.dot(h1, w2_ref[...], preferred_element_type=jnp.float32)
  xw2_ref[...] = xw2
  y2_ref[...] = d * xw2


def _tc3_body(z2_ref, xw2_ref, d_ref, b2_ref, wf1_ref, bf1_ref, wf2_ref,
              bf2_ref, out_ref):
  d = d_ref[...]
  h2 = jnp.maximum(d * z2_ref[...] + (d * d) * xw2_ref[...] + b2_ref[...], 0.0)
  h3 = jnp.maximum(jnp.dot(h2, wf1_ref[...],
                           preferred_element_type=jnp.float32) + bf1_ref[...],
                   0.0)
  out_ref[...] = jnp.dot(h3, wf2_ref[...],
                         preferred_element_type=jnp.float32) + bf2_ref[...]


def _row_spec(cols):
  return pl.BlockSpec((_BN, cols), lambda i: (i, 0))


def _full_spec(r, c):
  return pl.BlockSpec((r, c), lambda i: (0, 0))


def _unquarter(a):
  return jnp.concatenate([a[q * QPAD:q * QPAD + QUARTER] for q in range(NQ)])


def kernel(x, edge_index, W1, b1, W2, b2, Wf1, bf1, Wf2, bf2):
  ei = edge_index.astype(jnp.int32)
  src, dst = ei[0], ei[1]
  e = src.shape[0]
  ep = ((e + NS * CHUNK - 1) // (NS * CHUNK)) * (NS * CHUNK)
  pad = ep - e
  src_p = jnp.concatenate([src, jnp.zeros((pad,), jnp.int32)])
  dst_p = jnp.concatenate([dst, jnp.full((pad,), N, jnp.int32)])
  src2 = src_p.reshape(ep // 128, 128)
  dst2 = dst_p.reshape(ep // 128, 128)
  nrows = ep // 128

  deg2 = _make_deg_kernel(nrows)(dst2, jnp.zeros((QTROWS,), jnp.float32))
  deg = _unquarter(deg2).reshape(N, 1)

  grid = (pl.cdiv(N, _BN),)
  d, dx = pl.pallas_call(
      _tc1_body,
      grid=grid,
      in_specs=[_row_spec(1), _row_spec(4)],
      out_specs=[_row_spec(1), _row_spec(4)],
      out_shape=[
          jax.ShapeDtypeStruct((N, 1), jnp.float32),
          jax.ShapeDtypeStruct((N, 4), jnp.float32),
      ],
  )(deg, x)

  scat4 = _make_scatter_kernel(nrows, 4)
  z1c = scat4(src2, dst2, dx, jnp.zeros((ZROWS, 4), jnp.float32))
  z1 = _unquarter(z1c)

  xw2, y2 = pl.pallas_call(
      _tc2_body,
      grid=grid,
      in_specs=[_row_spec(4), _row_spec(4), _row_spec(1),
                _full_spec(4, 32), _full_spec(1, 32), _full_spec(32, 32)],
      out_specs=[_row_spec(32), _row_spec(32)],
      out_shape=[
          jax.ShapeDtypeStruct((N, 32), jnp.float32),
          jax.ShapeDtypeStruct((N, 32), jnp.float32),
      ],
  )(z1, x, d, W1, b1.reshape(1, 32), W2)

  scat32 = _make_scatter_kernel(nrows, 32)
  z2c = scat32(src2, dst2, y2, jnp.zeros((ZROWS, 32), jnp.float32))
  z2 = _unquarter(z2c)

  out = pl.pallas_call(
      _tc3_body,
      grid=grid,
      in_specs=[_row_spec(32), _row_spec(32), _row_spec(1),
                _full_spec(1, 32), _full_spec(32, 64), _full_spec(1, 64),
                _full_spec(64, 2), _full_spec(1, 2)],
      out_specs=_row_spec(2),
      out_shape=jax.ShapeDtypeStruct((N, 2), jnp.float32),
  )(z2, xw2, d, b2.reshape(1, 32), Wf1, bf1.reshape(1, 64), Wf2,
    bf2.reshape(1, 2))
  return out


# speedup vs baseline: 27.9788x; 1.5631x over previous
import functools

import jax
import jax.numpy as jnp
from jax import lax
from jax.experimental import pallas as pl
from jax.experimental.pallas import tpu as pltpu
from jax.experimental.pallas import tpu_sc as plsc

N = 100000
NC = 2
NS = 16
LANE = 16
NQ = 4
QUARTER = N // NQ
QTROWS = 1568
QPAD = NS * QTROWS
KMAX = 16

_MESH = plsc.VectorSubcoreMesh(
    core_axis_name="c", subcore_axis_name="s", num_cores=NC, num_subcores=NS
)
_SC_PARAMS = pltpu.CompilerParams(use_tc_tiling_on_sc=False)


def _localize(sraw, draw, smb, dmb, a, base, k):
  for j in range(k):
    for l in range(128 // LANE):
      sl = pl.ds(l * LANE, LANE)
      fl = pl.ds(j * 128 + l * LANE, LANE)
      dl = draw[a, j, sl] - base
      m = dl.astype(jnp.uint32) < jnp.uint32(QUARTER)
      if smb is not None:
        smb[a, fl] = jnp.where(m, sraw[a, j, sl], -1)
      dmb[a, fl] = jnp.where(m, dl, -1)


def _make_deg_kernel(nrows_total, k):
  rpt = nrows_total // NS
  npair = rpt // (2 * k)

  def body(dst2, zeros1, deg_out, draw, dmb, ones, zbuf, acc,
           isem_a, isem_b, ssem_a, ssem_b):
    c = lax.axis_index("c")
    s = lax.axis_index("s")
    for v in range(k * 128 // LANE):
      ones[pl.ds(v * LANE, LANE)] = jnp.ones((LANE,), jnp.float32)
    pltpu.sync_copy(zeros1, zbuf)

    def scat(a, sem):
      return pltpu.async_copy(
          ones, acc.at[plsc.Indices(dmb.at[a], ignored_value=-1)],
          sem, add=True)

    for p in range(NQ // NC):
      q = p * NC + c
      base = q * QUARTER
      pltpu.sync_copy(zbuf, acc.at[pl.ds(s * QTROWS, QTROWS)])
      plsc.subcore_barrier()

      def pair(i, _):
        row0 = s * rpt + i * (2 * k)
        pltpu.async_copy(dst2.at[pl.ds(row0, k)], draw.at[0], isem_a)
        pltpu.async_copy(dst2.at[pl.ds(row0 + k, k)], draw.at[1], isem_b)

        @pl.when(i > 0)
        def _drain_b():
          scat_b_desc = pltpu.make_async_copy(
              ones, acc.at[plsc.Indices(dmb.at[1], ignored_value=-1)], ssem_b)
          scat_b_desc.wait()

        pltpu.make_async_copy(
            dst2.at[pl.ds(row0, k)], draw.at[0], isem_a).wait()
        _localize(None, draw, None, dmb, 0, base, k)
        scat(0, ssem_a)
        pltpu.make_async_copy(
            dst2.at[pl.ds(row0 + k, k)], draw.at[1], isem_b).wait()
        _localize(None, draw, None, dmb, 1, base, k)
        scat(1, ssem_b)
        pltpu.make_async_copy(
            ones, acc.at[plsc.Indices(dmb.at[0], ignored_value=-1)],
            ssem_a).wait()
        return _

      lax.fori_loop(0, npair, pair, 0)
      pltpu.make_async_copy(
          ones, acc.at[plsc.Indices(dmb.at[1], ignored_value=-1)],
          ssem_b).wait()
      plsc.subcore_barrier()
      off = pl.multiple_of(q * QPAD + s * QTROWS, 8)
      pltpu.sync_copy(acc.at[pl.ds(s * QTROWS, QTROWS)], zbuf)
      pltpu.sync_copy(zbuf, deg_out.at[pl.ds(off, QTROWS)])
      if p == 0:
        pltpu.sync_copy(zeros1, zbuf)

  return pl.kernel(
      body,
      out_type=jax.ShapeDtypeStruct((NQ * QPAD,), jnp.float32),
      mesh=_MESH,
      compiler_params=_SC_PARAMS,
      scratch_types=[
          pltpu.VMEM((2, k, 128), jnp.int32),
          pltpu.VMEM((2, k * 128), jnp.int32),
          pltpu.VMEM((k * 128,), jnp.float32),
          pltpu.VMEM((QTROWS,), jnp.float32),
          pltpu.VMEM_SHARED((QPAD,), jnp.float32),
          pltpu.SemaphoreType.DMA,
          pltpu.SemaphoreType.DMA,
          pltpu.SemaphoreType.DMA,
          pltpu.SemaphoreType.DMA,
      ],
  )


def _make_scatter_kernel(nrows_total, feat, k, zrows):
  rpt = nrows_total // NS
  npair = rpt // (2 * k)

  def body(src2, dst2, y, zeros2, z_out, sraw, draw, smb, dmb, zbuf, rows,
           acc, isem_a, isem_b, gsem_a, gsem_b, ssem_a, ssem_b):
    c = lax.axis_index("c")
    s = lax.axis_index("s")

    def gath(a, sem):
      return pltpu.async_copy(
          y.at[plsc.Indices(smb.at[a], ignored_value=-1)], rows.at[a], sem)

    def scat(a, sem):
      return pltpu.async_copy(
          rows.at[a], acc.at[plsc.Indices(dmb.at[a], ignored_value=-1)],
          sem, add=True)

    for p in range(NQ // NC):
      q = p * NC + c
      base = q * QUARTER
      pltpu.sync_copy(zeros2, zbuf)
      for r in range(QTROWS // zrows):
        pltpu.sync_copy(zbuf, acc.at[pl.ds(s * QTROWS + r * zrows, zrows)])
      plsc.subcore_barrier()

      def pair(i, _):
        row0 = s * rpt + i * (2 * k)
        pltpu.async_copy(src2.at[pl.ds(row0, k)], sraw.at[0], isem_a)
        pltpu.async_copy(dst2.at[pl.ds(row0, k)], draw.at[0], isem_a)
        pltpu.async_copy(src2.at[pl.ds(row0 + k, k)], sraw.at[1], isem_b)
        pltpu.async_copy(dst2.at[pl.ds(row0 + k, k)], draw.at[1], isem_b)

        @pl.when(i > 0)
        def _drain_b():
          pltpu.make_async_copy(
              rows.at[1], acc.at[plsc.Indices(dmb.at[1], ignored_value=-1)],
              ssem_b).wait()

        pltpu.make_async_copy(
            src2.at[pl.ds(row0, k)], sraw.at[0], isem_a).wait()
        pltpu.make_async_copy(
            dst2.at[pl.ds(row0, k)], draw.at[0], isem_a).wait()
        _localize(sraw, draw, smb, dmb, 0, base, k)
        gath(0, gsem_a)
        pltpu.make_async_copy(
            src2.at[pl.ds(row0 + k, k)], sraw.at[1], isem_b).wait()
        pltpu.make_async_copy(
            dst2.at[pl.ds(row0 + k, k)], draw.at[1], isem_b).wait()
        _localize(sraw, draw, smb, dmb, 1, base, k)
        gath(1, gsem_b)
        pltpu.make_async_copy(
            y.at[plsc.Indices(smb.at[0], ignored_value=-1)], rows.at[0],
            gsem_a).wait()
        scat(0, ssem_a)
        pltpu.make_async_copy(
            y.at[plsc.Indices(smb.at[1], ignored_value=-1)], rows.at[1],
            gsem_b).wait()
        scat(1, ssem_b)
        pltpu.make_async_copy(
            rows.at[0], acc.at[plsc.Indices(dmb.at[0], ignored_value=-1)],
            ssem_a).wait()
        return _

      lax.fori_loop(0, npair, pair, 0)
      pltpu.make_async_copy(
          rows.at[1], acc.at[plsc.Indices(dmb.at[1], ignored_value=-1)],
          ssem_b).wait()
      plsc.subcore_barrier()
      off = pl.multiple_of(q * QPAD + s * QTROWS, 8)
      for r in range(QTROWS // zrows):
        pltpu.sync_copy(acc.at[pl.ds(s * QTROWS + r * zrows, zrows)], zbuf)
        pltpu.sync_copy(zbuf, z_out.at[pl.ds(off + r * zrows, zrows)])

  return pl.kernel(
      body,
      out_type=jax.ShapeDtypeStruct((NQ * QPAD, feat), jnp.float32),
      mesh=_MESH,
      compiler_params=_SC_PARAMS,
      scratch_types=[
          pltpu.VMEM((2, k, 128), jnp.int32),
          pltpu.VMEM((2, k, 128), jnp.int32),
          pltpu.VMEM((2, k * 128), jnp.int32),
          pltpu.VMEM((2, k * 128), jnp.int32),
          pltpu.VMEM((zrows, feat), jnp.float32),
          pltpu.VMEM((2, k * 128, feat), jnp.float32),
          pltpu.VMEM_SHARED((QPAD, feat), jnp.float32),
          pltpu.SemaphoreType.DMA,
          pltpu.SemaphoreType.DMA,
          pltpu.SemaphoreType.DMA,
          pltpu.SemaphoreType.DMA,
          pltpu.SemaphoreType.DMA,
          pltpu.SemaphoreType.DMA,
      ],
  )


_BN = 8192


def _tc2_body(z1_ref, x_ref, d_ref, w1_ref, b1_ref, w2_ref, xw2_ref, y2_ref):
  d = d_ref[...]
  u = d * z1_ref[...] + (d * d) * x_ref[...]
  h1 = jnp.maximum(jnp.dot(u, w1_ref[...],
                           preferred_element_type=jnp.float32) + b1_ref[...],
                   0.0)
  xw2 = jnp.dot(h1, w2_ref[...], preferred_element_type=jnp.float32)
  xw2_ref[...] = xw2
  y2_ref[...] = d * xw2


def _tc3_body(z2_ref, xw2_ref, d_ref, b2_ref, wf1_ref, bf1_ref, wf2_ref,
              bf2_ref, out_ref):
  d = d_ref[...]
  h2 = jnp.maximum(d * z2_ref[...] + (d * d) * xw2_ref[...] + b2_ref[...], 0.0)
  h3 = jnp.maximum(jnp.dot(h2, wf1_ref[...],
                           preferred_element_type=jnp.float32) + bf1_ref[...],
                   0.0)
  out_ref[...] = jnp.dot(h3, wf2_ref[...],
                         preferred_element_type=jnp.float32) + bf2_ref[...]


def _row_spec(cols):
  return pl.BlockSpec((_BN, cols), lambda i: (i, 0))


def _full_spec(r, c):
  return pl.BlockSpec((r, c), lambda i: (0, 0))


def _unquarter(a):
  return jnp.concatenate([a[q * QPAD:q * QPAD + QUARTER] for q in range(NQ)])


def kernel(x, edge_index, W1, b1, W2, b2, Wf1, bf1, Wf2, bf2):
  ei = edge_index.astype(jnp.int32)
  src, dst = ei[0], ei[1]
  e = src.shape[0]
  ep = ((e + NS * KMAX * 128 - 1) // (NS * KMAX * 128)) * (NS * KMAX * 128)
  pad = ep - e
  src_p = jnp.concatenate([src, jnp.zeros((pad,), jnp.int32)])
  dst_p = jnp.concatenate([dst, jnp.full((pad,), N, jnp.int32)])
  src2 = src_p.reshape(ep // 128, 128)
  dst2 = dst_p.reshape(ep // 128, 128)
  nrows = ep // 128

  deg2 = _make_deg_kernel(nrows, 14)(dst2, jnp.zeros((QTROWS,), jnp.float32))
  deg = _unquarter(deg2).reshape(N, 1)
  d = lax.rsqrt(deg + 1.0)
  dx = x * d

  scat4 = _make_scatter_kernel(nrows, 4, 14, 392)
  z1c = scat4(src2, dst2, dx, jnp.zeros((392, 4), jnp.float32))
  z1 = _unquarter(z1c)

  grid = (pl.cdiv(N, _BN),)
  xw2, y2 = pl.pallas_call(
      _tc2_body,
      grid=grid,
      in_specs=[_row_spec(4), _row_spec(4), _row_spec(1),
                _full_spec(4, 32), _full_spec(1, 32), _full_spec(32, 32)],
      out_specs=[_row_spec(32), _row_spec(32)],
      out_shape=[
          jax.ShapeDtypeStruct((N, 32), jnp.float32),
          jax.ShapeDtypeStruct((N, 32), jnp.float32),
      ],
  )(z1, x, d, W1, b1.reshape(1, 32), W2)

  scat32 = _make_scatter_kernel(nrows, 32, 7, 196)
  z2c = scat32(src2, dst2, y2, jnp.zeros((196, 32), jnp.float32))
  z2 = _unquarter(z2c)

  out = pl.pallas_call(
      _tc3_body,
      grid=grid,
      in_specs=[_row_spec(32), _row_spec(32), _row_spec(1),
                _full_spec(1, 32), _full_spec(32, 64), _full_spec(1, 64),
                _full_spec(64, 2), _full_spec(1, 2)],
      out_specs=_row_spec(2),
      out_shape=jax.ShapeDtypeStruct((N, 2), jnp.float32),
  )(z2, xw2, d, b2.reshape(1, 32), Wf1, bf1.reshape(1, 64), Wf2,
    bf2.reshape(1, 2))
  return out


# speedup vs baseline: 29.9720x; 1.0712x over previous
import functools

import jax
import jax.numpy as jnp
from jax import lax
from jax.experimental import pallas as pl
from jax.experimental.pallas import tpu as pltpu
from jax.experimental.pallas import tpu_sc as plsc

N = 100000
NC = 2
NS = 16
LANE = 16
NQ = 4
QTROWS = 1568
QUARTER = NS * QTROWS
NP = NQ * QUARTER
KMAX = 16

_MESH = plsc.VectorSubcoreMesh(
    core_axis_name="c", subcore_axis_name="s", num_cores=NC, num_subcores=NS
)
_SC_PARAMS = pltpu.CompilerParams(use_tc_tiling_on_sc=False)


def _localize(sraw, draw, smb, dmb, a, base, k):
  for j in range(k):
    for l in range(128 // LANE):
      sl = pl.ds(l * LANE, LANE)
      fl = pl.ds(j * 128 + l * LANE, LANE)
      dl = draw[a, j, sl] - base
      m = dl.astype(jnp.uint32) < jnp.uint32(QUARTER)
      if smb is not None:
        smb[a, fl] = jnp.where(m, sraw[a, j, sl], -1)
      dmb[a, fl] = jnp.where(m, dl, -1)


def _make_deg_kernel(nrows_total, k):
  rpt = nrows_total // NS
  npair = rpt // (2 * k)

  def body(dst2, zeros1, deg_out, draw, dmb, ones, zbuf, acc,
           isem_a, isem_b, ssem_a, ssem_b):
    c = lax.axis_index("c")
    s = lax.axis_index("s")
    for v in range(k * 128 // LANE):
      ones[pl.ds(v * LANE, LANE)] = jnp.ones((LANE,), jnp.float32)
    pltpu.sync_copy(zeros1, zbuf)

    def scat(a, sem):
      return pltpu.async_copy(
          ones, acc.at[plsc.Indices(dmb.at[a], ignored_value=-1)],
          sem, add=True)

    for p in range(NQ // NC):
      q = p * NC + c
      base = q * QUARTER
      pltpu.sync_copy(zbuf, acc.at[pl.ds(s * QTROWS, QTROWS)])
      plsc.subcore_barrier()

      def pair(i, _):
        row0 = s * rpt + i * (2 * k)
        pltpu.async_copy(dst2.at[pl.ds(row0, k)], draw.at[0], isem_a)
        pltpu.async_copy(dst2.at[pl.ds(row0 + k, k)], draw.at[1], isem_b)

        @pl.when(i > 0)
        def _drain_b():
          scat_b_desc = pltpu.make_async_copy(
              ones, acc.at[plsc.Indices(dmb.at[1], ignored_value=-1)], ssem_b)
          scat_b_desc.wait()

        pltpu.make_async_copy(
            dst2.at[pl.ds(row0, k)], draw.at[0], isem_a).wait()
        _localize(None, draw, None, dmb, 0, base, k)
        scat(0, ssem_a)
        pltpu.make_async_copy(
            dst2.at[pl.ds(row0 + k, k)], draw.at[1], isem_b).wait()
        _localize(None, draw, None, dmb, 1, base, k)
        scat(1, ssem_b)
        pltpu.make_async_copy(
            ones, acc.at[plsc.Indices(dmb.at[0], ignored_value=-1)],
            ssem_a).wait()
        return _

      lax.fori_loop(0, npair, pair, 0)
      pltpu.make_async_copy(
          ones, acc.at[plsc.Indices(dmb.at[1], ignored_value=-1)],
          ssem_b).wait()
      plsc.subcore_barrier()
      off = pl.multiple_of(q * QUARTER + s * QTROWS, 8)
      pltpu.sync_copy(acc.at[pl.ds(s * QTROWS, QTROWS)], zbuf)
      pltpu.sync_copy(zbuf, deg_out.at[pl.ds(off, QTROWS)])
      if p == 0:
        pltpu.sync_copy(zeros1, zbuf)

  return pl.kernel(
      body,
      out_type=jax.ShapeDtypeStruct((NP,), jnp.float32),
      mesh=_MESH,
      compiler_params=_SC_PARAMS,
      scratch_types=[
          pltpu.VMEM((2, k, 128), jnp.int32),
          pltpu.VMEM((2, k * 128), jnp.int32),
          pltpu.VMEM((k * 128,), jnp.float32),
          pltpu.VMEM((QTROWS,), jnp.float32),
          pltpu.VMEM_SHARED((QUARTER,), jnp.float32),
          pltpu.SemaphoreType.DMA,
          pltpu.SemaphoreType.DMA,
          pltpu.SemaphoreType.DMA,
          pltpu.SemaphoreType.DMA,
      ],
  )


def _make_scatter_kernel(nrows_total, feat, k, zrows):
  rpt = nrows_total // NS
  npair = rpt // (2 * k)

  def body(src2, dst2, y, zeros2, z_out, sraw, draw, smb, dmb, zbuf, rows,
           acc, isem_a, isem_b, gsem_a, gsem_b, ssem_a, ssem_b):
    c = lax.axis_index("c")
    s = lax.axis_index("s")

    def gath(a, sem):
      return pltpu.async_copy(
          y.at[plsc.Indices(smb.at[a], ignored_value=-1)], rows.at[a], sem)

    def scat(a, sem):
      return pltpu.async_copy(
          rows.at[a], acc.at[plsc.Indices(dmb.at[a], ignored_value=-1)],
          sem, add=True)

    for p in range(NQ // NC):
      q = p * NC + c
      base = q * QUARTER
      pltpu.sync_copy(zeros2, zbuf)
      for r in range(QTROWS // zrows):
        pltpu.sync_copy(zbuf, acc.at[pl.ds(s * QTROWS + r * zrows, zrows)])
      plsc.subcore_barrier()

      def pair(i, _):
        row0 = s * rpt + i * (2 * k)
        pltpu.async_copy(src2.at[pl.ds(row0, k)], sraw.at[0], isem_a)
        pltpu.async_copy(dst2.at[pl.ds(row0, k)], draw.at[0], isem_a)
        pltpu.async_copy(src2.at[pl.ds(row0 + k, k)], sraw.at[1], isem_b)
        pltpu.async_copy(dst2.at[pl.ds(row0 + k, k)], draw.at[1], isem_b)

        @pl.when(i > 0)
        def _drain_b():
          pltpu.make_async_copy(
              rows.at[1], acc.at[plsc.Indices(dmb.at[1], ignored_value=-1)],
              ssem_b).wait()

        pltpu.make_async_copy(
            src2.at[pl.ds(row0, k)], sraw.at[0], isem_a).wait()
        pltpu.make_async_copy(
            dst2.at[pl.ds(row0, k)], draw.at[0], isem_a).wait()
        _localize(sraw, draw, smb, dmb, 0, base, k)
        gath(0, gsem_a)
        pltpu.make_async_copy(
            src2.at[pl.ds(row0 + k, k)], sraw.at[1], isem_b).wait()
        pltpu.make_async_copy(
            dst2.at[pl.ds(row0 + k, k)], draw.at[1], isem_b).wait()
        _localize(sraw, draw, smb, dmb, 1, base, k)
        gath(1, gsem_b)
        pltpu.make_async_copy(
            y.at[plsc.Indices(smb.at[0], ignored_value=-1)], rows.at[0],
            gsem_a).wait()
        scat(0, ssem_a)
        pltpu.make_async_copy(
            y.at[plsc.Indices(smb.at[1], ignored_value=-1)], rows.at[1],
            gsem_b).wait()
        scat(1, ssem_b)
        pltpu.make_async_copy(
            rows.at[0], acc.at[plsc.Indices(dmb.at[0], ignored_value=-1)],
            ssem_a).wait()
        return _

      lax.fori_loop(0, npair, pair, 0)
      pltpu.make_async_copy(
          rows.at[1], acc.at[plsc.Indices(dmb.at[1], ignored_value=-1)],
          ssem_b).wait()
      plsc.subcore_barrier()
      off = pl.multiple_of(q * QUARTER + s * QTROWS, 8)
      for r in range(QTROWS // zrows):
        pltpu.sync_copy(acc.at[pl.ds(s * QTROWS + r * zrows, zrows)], zbuf)
        pltpu.sync_copy(zbuf, z_out.at[pl.ds(off + r * zrows, zrows)])

  return pl.kernel(
      body,
      out_type=jax.ShapeDtypeStruct((NP, feat), jnp.float32),
      mesh=_MESH,
      compiler_params=_SC_PARAMS,
      scratch_types=[
          pltpu.VMEM((2, k, 128), jnp.int32),
          pltpu.VMEM((2, k, 128), jnp.int32),
          pltpu.VMEM((2, k * 128), jnp.int32),
          pltpu.VMEM((2, k * 128), jnp.int32),
          pltpu.VMEM((zrows, feat), jnp.float32),
          pltpu.VMEM((2, k * 128, feat), jnp.float32),
          pltpu.VMEM_SHARED((QUARTER, feat), jnp.float32),
          pltpu.SemaphoreType.DMA,
          pltpu.SemaphoreType.DMA,
          pltpu.SemaphoreType.DMA,
          pltpu.SemaphoreType.DMA,
          pltpu.SemaphoreType.DMA,
          pltpu.SemaphoreType.DMA,
      ],
  )


_BN = 8192


def _tcd_body(deg_ref, x_ref, dx_ref):
  d = lax.rsqrt(deg_ref[...] + 1.0)
  dx_ref[...] = x_ref[...] * d


def _tc2_body(z1_ref, x_ref, deg_ref, w1_ref, b1_ref, w2_ref, xw2_ref,
              y2_ref):
  d = lax.rsqrt(deg_ref[...] + 1.0)
  u = d * z1_ref[...] + (d * d) * x_ref[...]
  h1 = jnp.maximum(jnp.dot(u, w1_ref[...],
                           preferred_element_type=jnp.float32) + b1_ref[...],
                   0.0)
  xw2 = jnp.dot(h1, w2_ref[...], preferred_element_type=jnp.float32)
  xw2_ref[...] = xw2
  y2_ref[...] = d * xw2


def _tc3_body(z2_ref, xw2_ref, deg_ref, b2_ref, wf1_ref, bf1_ref, wf2_ref,
              bf2_ref, out_ref):
  d = lax.rsqrt(deg_ref[...] + 1.0)
  h2 = jnp.maximum(d * z2_ref[...] + (d * d) * xw2_ref[...] + b2_ref[...], 0.0)
  h3 = jnp.maximum(jnp.dot(h2, wf1_ref[...],
                           preferred_element_type=jnp.float32) + bf1_ref[...],
                   0.0)
  out_ref[...] = jnp.dot(h3, wf2_ref[...],
                         preferred_element_type=jnp.float32) + bf2_ref[...]


def _row_spec(cols):
  return pl.BlockSpec((_BN, cols), lambda i: (i, 0))


def _full_spec(r, c):
  return pl.BlockSpec((r, c), lambda i: (0, 0))


def kernel(x, edge_index, W1, b1, W2, b2, Wf1, bf1, Wf2, bf2):
  ei = edge_index.astype(jnp.int32)
  src, dst = ei[0], ei[1]
  e = src.shape[0]
  ep = ((e + NS * KMAX * 128 - 1) // (NS * KMAX * 128)) * (NS * KMAX * 128)
  pad = ep - e
  src_p = jnp.concatenate([src, jnp.zeros((pad,), jnp.int32)])
  dst_p = jnp.concatenate([dst, jnp.full((pad,), NP, jnp.int32)])
  src2 = src_p.reshape(ep // 128, 128)
  dst2 = dst_p.reshape(ep // 128, 128)
  nrows = ep // 128
  xp = jnp.pad(x, ((0, NP - N), (0, 0)))

  deg2 = _make_deg_kernel(nrows, 14)(dst2, jnp.zeros((QTROWS,), jnp.float32))
  deg = deg2.reshape(NP, 1)

  grid = (pl.cdiv(NP, _BN),)
  dx = pl.pallas_call(
      _tcd_body,
      grid=grid,
      in_specs=[_row_spec(1), _row_spec(4)],
      out_specs=_row_spec(4),
      out_shape=jax.ShapeDtypeStruct((NP, 4), jnp.float32),
  )(deg, xp)

  scat4 = _make_scatter_kernel(nrows, 4, 14, 392)
  z1 = scat4(src2, dst2, dx, jnp.zeros((392, 4), jnp.float32))

  xw2, y2 = pl.pallas_call(
      _tc2_body,
      grid=grid,
      in_specs=[_row_spec(4), _row_spec(4), _row_spec(1),
                _full_spec(4, 32), _full_spec(1, 32), _full_spec(32, 32)],
      out_specs=[_row_spec(32), _row_spec(32)],
      out_shape=[
          jax.ShapeDtypeStruct((NP, 32), jnp.float32),
          jax.ShapeDtypeStruct((NP, 32), jnp.float32),
      ],
  )(z1, xp, deg, W1, b1.reshape(1, 32), W2)

  scat32 = _make_scatter_kernel(nrows, 32, 7, 196)
  z2 = scat32(src2, dst2, y2, jnp.zeros((196, 32), jnp.float32))

  out = pl.pallas_call(
      _tc3_body,
      grid=grid,
      in_specs=[_row_spec(32), _row_spec(32), _row_spec(1),
                _full_spec(1, 32), _full_spec(32, 64), _full_spec(1, 64),
                _full_spec(64, 2), _full_spec(1, 2)],
      out_specs=_row_spec(2),
      out_shape=jax.ShapeDtypeStruct((NP, 2), jnp.float32),
  )(z2, xw2, deg, b2.reshape(1, 32), Wf1, bf1.reshape(1, 64), Wf2,
    bf2.reshape(1, 2))
  return out[:N]


# speedup vs baseline: 31.6461x; 1.0559x over previous
import functools

import jax
import jax.numpy as jnp
from jax import lax
from jax.experimental import pallas as pl
from jax.experimental.pallas import tpu as pltpu
from jax.experimental.pallas import tpu_sc as plsc

N = 100000
NC = 2
NS = 16
LANE = 16
NQ = 4
QTROWS = 1568
QUARTER = NS * QTROWS
NP = NQ * QUARTER
KMAX = 16

_MESH = plsc.VectorSubcoreMesh(
    core_axis_name="c", subcore_axis_name="s", num_cores=NC, num_subcores=NS
)
_SC_PARAMS = pltpu.CompilerParams(use_tc_tiling_on_sc=False)


def _localize(sraw, draw, smb, dmb, a, base, k, share):
  for j in range(k):
    for l in range(128 // LANE):
      sl = pl.ds(l * LANE, LANE)
      fl = pl.ds(j * 128 + l * LANE, LANE)
      dl = draw[a, j, sl] - base
      m = dl.astype(jnp.uint32) < jnp.uint32(share)
      if smb is not None:
        smb[a, fl] = jnp.where(m, sraw[a, j, sl], -1)
      dmb[a, fl] = jnp.where(m, dl, -1)


def _make_deg_kernel(nrows_total, k, nsplit):
  rpt = nrows_total // NS
  npair = rpt // (2 * k)
  share = NP // nsplit
  trows = share // NS

  def body(ei2, zeros1, deg_out, draw, dmb, ones, zbuf, acc,
           isem_a, isem_b, ssem_a, ssem_b):
    c = lax.axis_index("c")
    s = lax.axis_index("s")
    for v in range(k * 128 // LANE):
      ones[pl.ds(v * LANE, LANE)] = jnp.ones((LANE,), jnp.float32)
    pltpu.sync_copy(zeros1, zbuf)

    def scat(a, sem):
      return pltpu.async_copy(
          ones, acc.at[plsc.Indices(dmb.at[a], ignored_value=-1)],
          sem, add=True)

    for p in range(nsplit // NC):
      q = p * NC + c
      base = q * share
      pltpu.sync_copy(zbuf, acc.at[pl.ds(s * trows, trows)])
      plsc.subcore_barrier()

      def pair(i, _):
        row0 = s * rpt + i * (2 * k)
        pltpu.async_copy(ei2.at[1, pl.ds(row0, k)], draw.at[0], isem_a)
        pltpu.async_copy(ei2.at[1, pl.ds(row0 + k, k)], draw.at[1], isem_b)

        @pl.when(i > 0)
        def _drain_b():
          scat_b_desc = pltpu.make_async_copy(
              ones, acc.at[plsc.Indices(dmb.at[1], ignored_value=-1)], ssem_b)
          scat_b_desc.wait()

        pltpu.make_async_copy(
            ei2.at[1, pl.ds(row0, k)], draw.at[0], isem_a).wait()
        _localize(None, draw, None, dmb, 0, base, k, share)
        scat(0, ssem_a)
        pltpu.make_async_copy(
            ei2.at[1, pl.ds(row0 + k, k)], draw.at[1], isem_b).wait()
        _localize(None, draw, None, dmb, 1, base, k, share)
        scat(1, ssem_b)
        pltpu.make_async_copy(
            ones, acc.at[plsc.Indices(dmb.at[0], ignored_value=-1)],
            ssem_a).wait()
        return _

      lax.fori_loop(0, npair, pair, 0)
      pltpu.make_async_copy(
          ones, acc.at[plsc.Indices(dmb.at[1], ignored_value=-1)],
          ssem_b).wait()
      plsc.subcore_barrier()
      off = pl.multiple_of(q * share + s * trows, 8)
      pltpu.sync_copy(acc.at[pl.ds(s * trows, trows)], zbuf)
      pltpu.sync_copy(zbuf, deg_out.at[pl.ds(off, trows)])
      if p + 1 < nsplit // NC:
        pltpu.sync_copy(zeros1, zbuf)

  return pl.kernel(
      body,
      out_type=jax.ShapeDtypeStruct((NP,), jnp.float32),
      mesh=_MESH,
      compiler_params=_SC_PARAMS,
      scratch_types=[
          pltpu.VMEM((2, k, 128), jnp.int32),
          pltpu.VMEM((2, k * 128), jnp.int32),
          pltpu.VMEM((k * 128,), jnp.float32),
          pltpu.VMEM((share // NS,), jnp.float32),
          pltpu.VMEM_SHARED((share,), jnp.float32),
          pltpu.SemaphoreType.DMA,
          pltpu.SemaphoreType.DMA,
          pltpu.SemaphoreType.DMA,
          pltpu.SemaphoreType.DMA,
      ],
  )


def _make_scatter_kernel(nrows_total, feat, k, zrows, nsplit):
  rpt = nrows_total // NS
  npair = rpt // (2 * k)
  share = NP // nsplit
  trows = share // NS

  def body(ei2, y, zeros2, z_out, sraw, draw, smb, dmb, zbuf, rows,
           acc, isem_a, isem_b, gsem_a, gsem_b, ssem_a, ssem_b):
    c = lax.axis_index("c")
    s = lax.axis_index("s")

    def gath(a, sem):
      return pltpu.async_copy(
          y.at[plsc.Indices(smb.at[a], ignored_value=-1)], rows.at[a], sem)

    def scat(a, sem):
      return pltpu.async_copy(
          rows.at[a], acc.at[plsc.Indices(dmb.at[a], ignored_value=-1)],
          sem, add=True)

    for p in range(nsplit // NC):
      q = p * NC + c
      base = q * share
      pltpu.sync_copy(zeros2, zbuf)
      for r in range(trows // zrows):
        pltpu.sync_copy(zbuf, acc.at[pl.ds(s * trows + r * zrows, zrows)])
      plsc.subcore_barrier()

      def pair(i, _):
        row0 = s * rpt + i * (2 * k)
        pltpu.async_copy(ei2.at[0, pl.ds(row0, k)], sraw.at[0], isem_a)
        pltpu.async_copy(ei2.at[1, pl.ds(row0, k)], draw.at[0], isem_a)
        pltpu.async_copy(ei2.at[0, pl.ds(row0 + k, k)], sraw.at[1], isem_b)
        pltpu.async_copy(ei2.at[1, pl.ds(row0 + k, k)], draw.at[1], isem_b)

        @pl.when(i > 0)
        def _drain_b():
          pltpu.make_async_copy(
              rows.at[1], acc.at[plsc.Indices(dmb.at[1], ignored_value=-1)],
              ssem_b).wait()

        pltpu.make_async_copy(
            ei2.at[0, pl.ds(row0, k)], sraw.at[0], isem_a).wait()
        pltpu.make_async_copy(
            ei2.at[1, pl.ds(row0, k)], draw.at[0], isem_a).wait()
        _localize(sraw, draw, smb, dmb, 0, base, k, share)
        gath(0, gsem_a)
        pltpu.make_async_copy(
            ei2.at[0, pl.ds(row0 + k, k)], sraw.at[1], isem_b).wait()
        pltpu.make_async_copy(
            ei2.at[1, pl.ds(row0 + k, k)], draw.at[1], isem_b).wait()
        _localize(sraw, draw, smb, dmb, 1, base, k, share)
        gath(1, gsem_b)
        pltpu.make_async_copy(
            y.at[plsc.Indices(smb.at[0], ignored_value=-1)], rows.at[0],
            gsem_a).wait()
        scat(0, ssem_a)
        pltpu.make_async_copy(
            y.at[plsc.Indices(smb.at[1], ignored_value=-1)], rows.at[1],
            gsem_b).wait()
        scat(1, ssem_b)
        pltpu.make_async_copy(
            rows.at[0], acc.at[plsc.Indices(dmb.at[0], ignored_value=-1)],
            ssem_a).wait()
        return _

      lax.fori_loop(0, npair, pair, 0)
      pltpu.make_async_copy(
          rows.at[1], acc.at[plsc.Indices(dmb.at[1], ignored_value=-1)],
          ssem_b).wait()
      plsc.subcore_barrier()
      off = pl.multiple_of(q * share + s * trows, 8)
      for r in range(trows // zrows):
        pltpu.sync_copy(acc.at[pl.ds(s * trows + r * zrows, zrows)], zbuf)
        pltpu.sync_copy(zbuf, z_out.at[pl.ds(off + r * zrows, zrows)])

  return pl.kernel(
      body,
      out_type=jax.ShapeDtypeStruct((NP, feat), jnp.float32),
      mesh=_MESH,
      compiler_params=_SC_PARAMS,
      scratch_types=[
          pltpu.VMEM((2, k, 128), jnp.int32),
          pltpu.VMEM((2, k, 128), jnp.int32),
          pltpu.VMEM((2, k * 128), jnp.int32),
          pltpu.VMEM((2, k * 128), jnp.int32),
          pltpu.VMEM((zrows, feat), jnp.float32),
          pltpu.VMEM((2, k * 128, feat), jnp.float32),
          pltpu.VMEM_SHARED((share, feat), jnp.float32),
          pltpu.SemaphoreType.DMA,
          pltpu.SemaphoreType.DMA,
          pltpu.SemaphoreType.DMA,
          pltpu.SemaphoreType.DMA,
          pltpu.SemaphoreType.DMA,
          pltpu.SemaphoreType.DMA,
      ],
  )


_BN = 8192


def _tcd_body(deg_ref, x_ref, dx_ref):
  d = lax.rsqrt(deg_ref[...] + 1.0)
  dx_ref[...] = x_ref[...] * d


def _tc2_body(z1_ref, x_ref, deg_ref, w1_ref, b1_ref, w2_ref, xw2_ref,
              y2_ref):
  d = lax.rsqrt(deg_ref[...] + 1.0)
  u = d * z1_ref[...] + (d * d) * x_ref[...]
  h1 = jnp.maximum(jnp.dot(u, w1_ref[...],
                           preferred_element_type=jnp.float32) + b1_ref[...],
                   0.0)
  xw2 = jnp.dot(h1, w2_ref[...], preferred_element_type=jnp.float32)
  xw2_ref[...] = xw2
  y2_ref[...] = d * xw2


def _tc3_body(z2_ref, xw2_ref, deg_ref, b2_ref, wf1_ref, bf1_ref, wf2_ref,
              bf2_ref, out_ref):
  d = lax.rsqrt(deg_ref[...] + 1.0)
  h2 = jnp.maximum(d * z2_ref[...] + (d * d) * xw2_ref[...] + b2_ref[...], 0.0)
  h3 = jnp.maximum(jnp.dot(h2, wf1_ref[...],
                           preferred_element_type=jnp.float32) + bf1_ref[...],
                   0.0)
  out_ref[...] = jnp.dot(h3, wf2_ref[...],
                         preferred_element_type=jnp.float32) + bf2_ref[...]


def _row_spec(cols):
  return pl.BlockSpec((_BN, cols), lambda i: (i, 0))


def _full_spec(r, c):
  return pl.BlockSpec((r, c), lambda i: (0, 0))


def kernel(x, edge_index, W1, b1, W2, b2, Wf1, bf1, Wf2, bf2):
  ei = edge_index.astype(jnp.int32)
  e = ei.shape[1]
  ep = ((e + NS * KMAX * 128 - 1) // (NS * KMAX * 128)) * (NS * KMAX * 128)
  ei2 = jnp.pad(ei, ((0, 0), (0, ep - e)),
                constant_values=NP).reshape(2, ep // 128, 128)
  nrows = ep // 128
  xp = jnp.pad(x, ((0, NP - N), (0, 0)))

  deg2 = _make_deg_kernel(nrows, 14, 4)(
      ei2, jnp.zeros((NP // 4 // NS,), jnp.float32))
  deg = deg2.reshape(NP, 1)

  grid = (pl.cdiv(NP, _BN),)
  dx = pl.pallas_call(
      _tcd_body,
      grid=grid,
      in_specs=[_row_spec(1), _row_spec(4)],
      out_specs=_row_spec(4),
      out_shape=jax.ShapeDtypeStruct((NP, 4), jnp.float32),
  )(deg, xp)

  scat4 = _make_scatter_kernel(nrows, 4, 14, 392, 4)
  z1 = scat4(ei2, dx, jnp.zeros((392, 4), jnp.float32))

  xw2, y2 = pl.pallas_call(
      _tc2_body,
      grid=grid,
      in_specs=[_row_spec(4), _row_spec(4), _row_spec(1),
                _full_spec(4, 32), _full_spec(1, 32), _full_spec(32, 32)],
      out_specs=[_row_spec(32), _row_spec(32)],
      out_shape=[
          jax.ShapeDtypeStruct((NP, 32), jnp.float32),
          jax.ShapeDtypeStruct((NP, 32), jnp.float32),
      ],
  )(z1, xp, deg, W1, b1.reshape(1, 32), W2)

  scat32 = _make_scatter_kernel(nrows, 32, 7, 196, 4)
  z2 = scat32(ei2, y2, jnp.zeros((196, 32), jnp.float32))

  out = pl.pallas_call(
      _tc3_body,
      grid=grid,
      in_specs=[_row_spec(32), _row_spec(32), _row_spec(1),
                _full_spec(1, 32), _full_spec(32, 64), _full_spec(1, 64),
                _full_spec(64, 2), _full_spec(1, 2)],
      out_specs=_row_spec(2),
      out_shape=jax.ShapeDtypeStruct((N, 2), jnp.float32),
  )(z2, xw2, deg, b2.reshape(1, 32), Wf1, bf1.reshape(1, 64), Wf2,
    bf2.reshape(1, 2))
  return out


# speedup vs baseline: 32.8578x; 1.0383x over previous
import functools

import jax
import jax.numpy as jnp
from jax import lax
from jax.experimental import pallas as pl
from jax.experimental.pallas import tpu as pltpu
from jax.experimental.pallas import tpu_sc as plsc

N = 100000
NC = 2
NS = 16
LANE = 16
NQ = 4
QTROWS = 1568
QUARTER = NS * QTROWS
NP = NQ * QUARTER
KMAX = 16

_MESH = plsc.VectorSubcoreMesh(
    core_axis_name="c", subcore_axis_name="s", num_cores=NC, num_subcores=NS
)
_SC_PARAMS = pltpu.CompilerParams(use_tc_tiling_on_sc=False)


def _localize(sraw, draw, smb, dmb, a, base, k, share):
  for j in range(k):
    for l in range(128 // LANE):
      sl = pl.ds(l * LANE, LANE)
      fl = pl.ds(j * 128 + l * LANE, LANE)
      dl = draw[a, j, sl] - base
      m = dl.astype(jnp.uint32) < jnp.uint32(share)
      if smb is not None:
        smb[a, fl] = jnp.where(m, sraw[a, j, sl], -1)
      dmb[a, fl] = jnp.where(m, dl, -1)


def _make_deg_kernel(nrows_total, k, nsplit):
  rpt = nrows_total // NS
  npair = rpt // (2 * k)
  share = NP // nsplit
  trows = share // NS

  def body(ei2, zeros1, deg_out, draw, dmb, ones, zbuf, acc,
           isem_a, isem_b, ssem_a, ssem_b):
    c = lax.axis_index("c")
    s = lax.axis_index("s")
    for v in range(k * 128 // LANE):
      ones[pl.ds(v * LANE, LANE)] = jnp.ones((LANE,), jnp.float32)
    pltpu.sync_copy(zeros1, zbuf)

    def scat(a, sem):
      return pltpu.async_copy(
          ones, acc.at[plsc.Indices(dmb.at[a], ignored_value=-1)],
          sem, add=True)

    for p in range(nsplit // NC):
      q = p * NC + c
      base = q * share
      pltpu.sync_copy(zbuf, acc.at[pl.ds(s * trows, trows)])
      plsc.subcore_barrier()

      def pair(i, _):
        row0 = s * rpt + i * (2 * k)
        pltpu.async_copy(ei2.at[1, pl.ds(row0, k)], draw.at[0], isem_a)
        pltpu.async_copy(ei2.at[1, pl.ds(row0 + k, k)], draw.at[1], isem_b)

        @pl.when(i > 0)
        def _drain_b():
          scat_b_desc = pltpu.make_async_copy(
              ones, acc.at[plsc.Indices(dmb.at[1], ignored_value=-1)], ssem_b)
          scat_b_desc.wait()

        pltpu.make_async_copy(
            ei2.at[1, pl.ds(row0, k)], draw.at[0], isem_a).wait()
        _localize(None, draw, None, dmb, 0, base, k, share)
        scat(0, ssem_a)
        pltpu.make_async_copy(
            ei2.at[1, pl.ds(row0 + k, k)], draw.at[1], isem_b).wait()
        _localize(None, draw, None, dmb, 1, base, k, share)
        scat(1, ssem_b)
        pltpu.make_async_copy(
            ones, acc.at[plsc.Indices(dmb.at[0], ignored_value=-1)],
            ssem_a).wait()
        return _

      lax.fori_loop(0, npair, pair, 0)
      pltpu.make_async_copy(
          ones, acc.at[plsc.Indices(dmb.at[1], ignored_value=-1)],
          ssem_b).wait()
      plsc.subcore_barrier()
      off = pl.multiple_of(q * share + s * trows, 8)
      pltpu.sync_copy(acc.at[pl.ds(s * trows, trows)], zbuf)
      pltpu.sync_copy(zbuf, deg_out.at[pl.ds(off, trows)])
      if p + 1 < nsplit // NC:
        pltpu.sync_copy(zeros1, zbuf)

  return pl.kernel(
      body,
      out_type=jax.ShapeDtypeStruct((NP,), jnp.float32),
      mesh=_MESH,
      compiler_params=_SC_PARAMS,
      scratch_types=[
          pltpu.VMEM((2, k, 128), jnp.int32),
          pltpu.VMEM((2, k * 128), jnp.int32),
          pltpu.VMEM((k * 128,), jnp.float32),
          pltpu.VMEM((share // NS,), jnp.float32),
          pltpu.VMEM_SHARED((share,), jnp.float32),
          pltpu.SemaphoreType.DMA,
          pltpu.SemaphoreType.DMA,
          pltpu.SemaphoreType.DMA,
          pltpu.SemaphoreType.DMA,
      ],
  )


def _make_scatter_kernel(nrows_total, feat, k, zrows, nsplit):
  rpt = nrows_total // NS
  npair = rpt // (2 * k)
  share = NP // nsplit
  trows = share // NS

  def body(ei2, y, zeros2, z_out, sraw, draw, smb, dmb, zbuf, rows,
           acc, isem_a, isem_b, gsem_a, gsem_b, ssem_a, ssem_b):
    c = lax.axis_index("c")
    s = lax.axis_index("s")

    def gath(a, sem):
      return pltpu.async_copy(
          y.at[plsc.Indices(smb.at[a], ignored_value=-1)], rows.at[a], sem)

    def scat(a, sem):
      return pltpu.async_copy(
          rows.at[a], acc.at[plsc.Indices(dmb.at[a], ignored_value=-1)],
          sem, add=True)

    for p in range(nsplit // NC):
      q = p * NC + c
      base = q * share
      pltpu.sync_copy(zeros2, zbuf)
      for r in range(trows // zrows):
        pltpu.sync_copy(zbuf, acc.at[pl.ds(s * trows + r * zrows, zrows)])
      plsc.subcore_barrier()

      def pair(i, _):
        row0 = s * rpt + i * (2 * k)
        pltpu.async_copy(ei2.at[0, pl.ds(row0, k)], sraw.at[0], isem_a)
        pltpu.async_copy(ei2.at[1, pl.ds(row0, k)], draw.at[0], isem_a)
        pltpu.async_copy(ei2.at[0, pl.ds(row0 + k, k)], sraw.at[1], isem_b)
        pltpu.async_copy(ei2.at[1, pl.ds(row0 + k, k)], draw.at[1], isem_b)

        @pl.when(i > 0)
        def _drain_b():
          pltpu.make_async_copy(
              rows.at[1], acc.at[plsc.Indices(dmb.at[1], ignored_value=-1)],
              ssem_b).wait()

        pltpu.make_async_copy(
            ei2.at[0, pl.ds(row0, k)], sraw.at[0], isem_a).wait()
        pltpu.make_async_copy(
            ei2.at[1, pl.ds(row0, k)], draw.at[0], isem_a).wait()
        _localize(sraw, draw, smb, dmb, 0, base, k, share)
        gath(0, gsem_a)
        pltpu.make_async_copy(
            ei2.at[0, pl.ds(row0 + k, k)], sraw.at[1], isem_b).wait()
        pltpu.make_async_copy(
            ei2.at[1, pl.ds(row0 + k, k)], draw.at[1], isem_b).wait()
        _localize(sraw, draw, smb, dmb, 1, base, k, share)
        gath(1, gsem_b)
        pltpu.make_async_copy(
            y.at[plsc.Indices(smb.at[0], ignored_value=-1)], rows.at[0],
            gsem_a).wait()
        scat(0, ssem_a)
        pltpu.make_async_copy(
            y.at[plsc.Indices(smb.at[1], ignored_value=-1)], rows.at[1],
            gsem_b).wait()
        scat(1, ssem_b)
        pltpu.make_async_copy(
            rows.at[0], acc.at[plsc.Indices(dmb.at[0], ignored_value=-1)],
            ssem_a).wait()
        return _

      lax.fori_loop(0, npair, pair, 0)
      pltpu.make_async_copy(
          rows.at[1], acc.at[plsc.Indices(dmb.at[1], ignored_value=-1)],
          ssem_b).wait()
      plsc.subcore_barrier()
      off = pl.multiple_of(q * share + s * trows, 8)
      for r in range(trows // zrows):
        pltpu.sync_copy(acc.at[pl.ds(s * trows + r * zrows, zrows)], zbuf)
        pltpu.sync_copy(zbuf, z_out.at[pl.ds(off + r * zrows, zrows)])

  return pl.kernel(
      body,
      out_type=jax.ShapeDtypeStruct((NP, feat), jnp.float32),
      mesh=_MESH,
      compiler_params=_SC_PARAMS,
      scratch_types=[
          pltpu.VMEM((2, k, 128), jnp.int32),
          pltpu.VMEM((2, k, 128), jnp.int32),
          pltpu.VMEM((2, k * 128), jnp.int32),
          pltpu.VMEM((2, k * 128), jnp.int32),
          pltpu.VMEM((zrows, feat), jnp.float32),
          pltpu.VMEM((2, k * 128, feat), jnp.float32),
          pltpu.VMEM_SHARED((share, feat), jnp.float32),
          pltpu.SemaphoreType.DMA,
          pltpu.SemaphoreType.DMA,
          pltpu.SemaphoreType.DMA,
          pltpu.SemaphoreType.DMA,
          pltpu.SemaphoreType.DMA,
          pltpu.SemaphoreType.DMA,
      ],
  )


_BN = 8192


def _tcd_body(deg_ref, x_ref, dx_ref):
  d = lax.rsqrt(deg_ref[...] + 1.0)
  dx_ref[...] = x_ref[...] * d


def _tc2_body(z1_ref, x_ref, deg_ref, w1_ref, b1_ref, w2_ref, xw2_ref,
              y2_ref):
  d = lax.rsqrt(deg_ref[...] + 1.0)
  u = d * z1_ref[...] + (d * d) * x_ref[...]
  h1 = jnp.maximum(jnp.dot(u, w1_ref[...],
                           preferred_element_type=jnp.float32) + b1_ref[...],
                   0.0)
  xw2 = jnp.dot(h1, w2_ref[...], preferred_element_type=jnp.float32)
  xw2_ref[...] = xw2
  y2_ref[...] = d * xw2


def _tc3_body(z2_ref, xw2_ref, deg_ref, b2_ref, wf1_ref, bf1_ref, wf2_ref,
              bf2_ref, out_ref):
  d = lax.rsqrt(deg_ref[...] + 1.0)
  h2 = jnp.maximum(d * z2_ref[...] + (d * d) * xw2_ref[...] + b2_ref[...], 0.0)
  h3 = jnp.maximum(jnp.dot(h2, wf1_ref[...],
                           preferred_element_type=jnp.float32) + bf1_ref[...],
                   0.0)
  out_ref[...] = jnp.dot(h3, wf2_ref[...],
                         preferred_element_type=jnp.float32) + bf2_ref[...]


def _row_spec(cols):
  return pl.BlockSpec((_BN, cols), lambda i: (i, 0))


def _full_spec(r, c):
  return pl.BlockSpec((r, c), lambda i: (0, 0))


def kernel(x, edge_index, W1, b1, W2, b2, Wf1, bf1, Wf2, bf2):
  ei = edge_index.astype(jnp.int32)
  e = ei.shape[1]
  ep = ((e + NS * KMAX * 128 - 1) // (NS * KMAX * 128)) * (NS * KMAX * 128)
  ei2 = jnp.pad(ei, ((0, 0), (0, ep - e)),
                constant_values=NP).reshape(2, ep // 128, 128)
  nrows = ep // 128
  xp = jnp.pad(x, ((0, NP - N), (0, 0)))

  deg2 = _make_deg_kernel(nrows, 14, 2)(
      ei2, jnp.zeros((NP // 2 // NS,), jnp.float32))
  deg = deg2.reshape(NP, 1)

  grid = (pl.cdiv(NP, _BN),)
  dx = pl.pallas_call(
      _tcd_body,
      grid=grid,
      in_specs=[_row_spec(1), _row_spec(4)],
      out_specs=_row_spec(4),
      out_shape=jax.ShapeDtypeStruct((NP, 4), jnp.float32),
  )(deg, xp)

  scat4 = _make_scatter_kernel(nrows, 4, 14, 392, 4)
  z1 = scat4(ei2, dx, jnp.zeros((392, 4), jnp.float32))

  xw2, y2 = pl.pallas_call(
      _tc2_body,
      grid=grid,
      in_specs=[_row_spec(4), _row_spec(4), _row_spec(1),
                _full_spec(4, 32), _full_spec(1, 32), _full_spec(32, 32)],
      out_specs=[_row_spec(32), _row_spec(32)],
      out_shape=[
          jax.ShapeDtypeStruct((NP, 32), jnp.float32),
          jax.ShapeDtypeStruct((NP, 32), jnp.float32),
      ],
  )(z1, xp, deg, W1, b1.reshape(1, 32), W2)

  scat32 = _make_scatter_kernel(nrows, 32, 7, 196, 4)
  z2 = scat32(ei2, y2, jnp.zeros((196, 32), jnp.float32))

  out = pl.pallas_call(
      _tc3_body,
      grid=grid,
      in_specs=[_row_spec(32), _row_spec(32), _row_spec(1),
                _full_spec(1, 32), _full_spec(32, 64), _full_spec(1, 64),
                _full_spec(64, 2), _full_spec(1, 2)],
      out_specs=_row_spec(2),
      out_shape=jax.ShapeDtypeStruct((N, 2), jnp.float32),
  )(z2, xw2, deg, b2.reshape(1, 32), Wf1, bf1.reshape(1, 64), Wf2,
    bf2.reshape(1, 2))
  return out
